# baseline (speedup 1.0000x reference)
import functools

import jax
import jax.numpy as jnp
from jax import lax
from jax.experimental import pallas as pl
from jax.experimental.pallas import tpu as pltpu
from jax.experimental.pallas import tpu_sc as plsc

NUM_STOCKS = 1000000
EMBED_DIM = 64
BATCH = 16384
SEQ_LEN = 50

NC = 2
NS = 16
NW = NC * NS

ROW_W = 2 * EMBED_DIM
BW = 128
B_PER_W = BATCH // NW
CPW = B_PER_W // BW
NUNIT = SEQ_LEN * CPW
NBUF = 2
L = 16
DG = EMBED_DIM // L
OSTRIDE = BW + 1


def _gather_kernel(ids_t, tab_w):
    mesh = plsc.VectorSubcoreMesh(core_axis_name="c", subcore_axis_name="s")

    @functools.partial(
        pl.kernel,
        mesh=mesh,
        out_type=jax.ShapeDtypeStruct((SEQ_LEN, EMBED_DIM, BATCH), jnp.float32),
        scratch_types=[
            pltpu.VMEM((SEQ_LEN, B_PER_W), jnp.int32),
            pltpu.VMEM((NBUF, BW), jnp.int32),
            pltpu.VMEM((NBUF, BW, ROW_W), jnp.float32),
            pltpu.VMEM((NBUF, EMBED_DIM, OSTRIDE), jnp.float32),
            pltpu.VMEM((NBUF, EMBED_DIM, BW), jnp.float32),
            [pltpu.SemaphoreType.DMA] * NBUF,
            [pltpu.SemaphoreType.DMA] * NBUF,
        ],
        compiler_params=pltpu.CompilerParams(needs_layout_passes=False),
    )
    def k(ids_hbm, tab_hbm, out_hbm, ids_v, idx_v, rows_v, skew_v, out_v,
          gsem, wsem):
        wid = lax.axis_index("s") * NC + lax.axis_index("c")
        b_base = wid * B_PER_W

        pltpu.sync_copy(ids_hbm.at[:, pl.ds(b_base, B_PER_W)], ids_v)

        def fire(u, b):
            s = u // CPW
            c = lax.rem(u, CPW)
            for g in range(BW // L):
                idx_v[b, pl.ds(g * L, L)] = ids_v[s, pl.ds(c * BW + g * L, L)]
            pltpu.async_copy(tab_hbm.at[idx_v.at[b]], rows_v.at[b], gsem[b])

        def gather_wait(b):
            pltpu.make_async_copy(
                tab_hbm.at[idx_v.at[b]], rows_v.at[b], gsem[b]).wait()

        def out_start(u, b):
            s = u // CPW
            c = lax.rem(u, CPW)
            pltpu.async_copy(
                out_v.at[b],
                out_hbm.at[s, :, pl.ds(b_base + c * BW, BW)],
                wsem[b],
            )

        def out_wait(u, b):
            s = u // CPW
            c = lax.rem(u, CPW)
            pltpu.make_async_copy(
                out_v.at[b],
                out_hbm.at[s, :, pl.ds(b_base + c * BW, BW)],
                wsem[b],
            ).wait()

        for b in range(NBUF):
            fire(b, b)

        rows_a = [lax.iota(jnp.int32, L) + dg * L for dg in range(DG)]
        iota_l = lax.iota(jnp.int32, L)

        @pl.loop(0, NUNIT, step=NBUF)
        def unit_loop(u0):
            for b in range(NBUF):
                u = u0 + b
                gather_wait(b)
                @pl.loop(0, BW, unroll=16)
                def pass_a(p):
                    pvec = jnp.full((L,), 0, jnp.int32) + p
                    for dg in range(DG):
                        v = rows_v[b, p, pl.ds(dg * L, L)]
                        plsc.store_scatter(
                            skew_v.at[b], [rows_a[dg], pvec], v)

                @pl.loop(0, EMBED_DIM, unroll=8)
                def pass_b(d):
                    dvec = jnp.full((L,), 0, jnp.int32) + d
                    for g in range(BW // L):
                        v = plsc.load_gather(
                            skew_v.at[b], [dvec, iota_l + g * L])
                        out_v[b, d, pl.ds(g * L, L)] = v
                @pl.when(u >= NBUF)
                def _():
                    out_wait(u - NBUF, b)
                out_start(u, b)
                @pl.when(u + NBUF < NUNIT)
                def _():
                    fire(u + NBUF, b)

        for b in range(NBUF):
            out_wait(NUNIT - NBUF + b, b)

    return k(ids_t, tab_w)


def kernel(stock_ids, table):
    ids_t = stock_ids.T.astype(jnp.int32)
    tab_w = jnp.concatenate([table, table], axis=1)
    out_p = _gather_kernel(ids_t, tab_w)
    return out_p.transpose(2, 0, 1)


# speedup vs baseline: 1.6543x; 1.6543x over previous
import functools

import jax
import jax.numpy as jnp
from jax import lax
from jax.experimental import pallas as pl
from jax.experimental.pallas import tpu as pltpu
from jax.experimental.pallas import tpu_sc as plsc

NUM_STOCKS = 1000000
EMBED_DIM = 64
BATCH = 16384
SEQ_LEN = 50
B_TOTAL = BATCH * SEQ_LEN

NC = 2
NS = 16
NW = NC * NS

IDX_W = 128
KROWS = 4
CHUNK = KROWS * IDX_W
ROWS_TOTAL = B_TOTAL // IDX_W
ROWS_PER_W = ROWS_TOTAL // NW
NCHUNK = ROWS_PER_W // KROWS
NBUF = 2


def _gather_kernel(ids2d, table):
    mesh = plsc.VectorSubcoreMesh(core_axis_name="c", subcore_axis_name="s")

    @functools.partial(
        pl.kernel,
        mesh=mesh,
        out_type=jax.ShapeDtypeStruct((B_TOTAL, EMBED_DIM), jnp.float32),
        scratch_types=[
            pltpu.VMEM((NBUF, KROWS, IDX_W), jnp.int32),
            pltpu.VMEM((NBUF, CHUNK, EMBED_DIM), jnp.float32),
            [pltpu.SemaphoreType.DMA] * NBUF,
        ],
        compiler_params=pltpu.CompilerParams(use_tc_tiling_on_sc=False),
    )
    def k(ids_hbm, table_hbm, out_hbm, idx_v, rows_v, sems):
        wid = lax.axis_index("s") * NC + lax.axis_index("c")
        row_base = wid * ROWS_PER_W

        def fire(i, b):
            row_off = row_base + i * KROWS
            pltpu.sync_copy(ids_hbm.at[pl.ds(row_off, KROWS)], idx_v.at[b])
            for j in range(KROWS):
                pltpu.async_copy(
                    table_hbm.at[idx_v.at[b, j]],
                    rows_v.at[b, pl.ds(j * IDX_W, IDX_W)],
                    sems[b],
                )

        def drain(b):
            for j in range(KROWS):
                pltpu.make_async_copy(
                    table_hbm.at[idx_v.at[b, j]],
                    rows_v.at[b, pl.ds(j * IDX_W, IDX_W)],
                    sems[b],
                ).wait()

        for b in range(NBUF):
            fire(b, b)

        @pl.loop(0, NCHUNK, step=NBUF)
        def chunk_loop(i):
            for b in range(NBUF):
                drain(b)
                row_off = row_base + (i + b) * KROWS
                pltpu.sync_copy(
                    rows_v.at[b], out_hbm.at[pl.ds(row_off * IDX_W, CHUNK)]
                )
                @pl.when(i + b + NBUF < NCHUNK)
                def _():
                    fire(i + b + NBUF, b)

    return k(ids2d, table)


def kernel(stock_ids, table):
    ids2d = stock_ids.reshape(ROWS_TOTAL, IDX_W).astype(jnp.int32)
    out = _gather_kernel(ids2d, table)
    return out.reshape(BATCH, SEQ_LEN, EMBED_DIM)
